# baseline (speedup 1.0000x reference)
import functools

import jax
import jax.numpy as jnp
from jax import lax
from jax.experimental import pallas as pl
from jax.experimental.pallas import tpu as pltpu
from jax.experimental.pallas import tpu_sc as plsc

_HIDDEN = 1024
_NC = 2
_NS = 16
_NW = _NC * _NS
_L = 16


@functools.cache
def _sc_lookup(total: int, n_rows: int):
    per_w = total // _NW
    mesh = plsc.VectorSubcoreMesh(core_axis_name="c", subcore_axis_name="s")

    @functools.partial(
        pl.kernel,
        out_type=jax.ShapeDtypeStruct((total, _HIDDEN), jnp.float32),
        mesh=mesh,
        compiler_params=pltpu.CompilerParams(
            use_tc_tiling_on_sc=False, needs_layout_passes=False
        ),
        scratch_types=[
            pltpu.VMEM((per_w,), jnp.int32),
            pltpu.VMEM((n_rows, _HIDDEN), jnp.float32),
            *[pltpu.VMEM((32, _HIDDEN), jnp.float32) for _ in range(3)],
            pltpu.SemaphoreType.DMA,
            pltpu.SemaphoreType.DMA,
            *[pltpu.SemaphoreType.DMA for _ in range(6)],
        ],
    )
    def k(ids_hbm, table_hbm, out_hbm, idx_v, table_v, b0, b1, b2,
          tsem, rsem, g0, g1, g2, s0, s1, s2):
        bufs = (b0, b1, b2)
        gsems = (g0, g1, g2)
        ssems = (s0, s1, s2)
        wid = lax.axis_index("s") * _NC + lax.axis_index("c")
        base = wid * per_w
        cp_t = pltpu.async_copy(table_hbm, table_v, tsem)
        pltpu.sync_copy(ids_hbm.at[wid], idx_v)
        gcp = [
            pltpu.async_copy(
                table_hbm.at[idx_v.at[pl.ds(c * 32, 32)]], bufs[c], gsems[c]
            )
            for c in range(3)
        ]
        cp_t.wait()
        iota = lax.iota(jnp.int32, _L)

        def fire(r, _):
            vec = idx_v[pl.ds((r // _L) * _L, _L)]
            rid = jnp.max(jnp.where(iota == r % _L, vec, 0))
            pltpu.async_copy(table_v.at[rid], out_hbm.at[base + r], rsem)
            return ()

        lax.fori_loop(96, per_w, fire, (), unroll=False)

        scp = []
        for c in range(3):
            gcp[c].wait()
            scp.append(pltpu.async_copy(
                bufs[c], out_hbm.at[pl.ds(base + c * 32, 32)], ssems[c]
            ))

        def drain(r, _):
            pltpu.make_async_copy(table_v.at[0], out_hbm.at[base], rsem).wait()
            return ()

        lax.fori_loop(96, per_w, drain, (), unroll=False)
        for cp in scp:
            cp.wait()

    return k


def kernel(ids, indicator_embs):
    b, s = ids.shape
    total = b * s
    ids_w = ids.astype(jnp.int32).reshape(_NW, total // _NW)
    out = _sc_lookup(total, indicator_embs.shape[0])(ids_w, indicator_embs)
    return out.reshape(b, s, _HIDDEN)
